# Initial kernel scaffold; baseline (speedup 1.0000x reference)
import jax
import jax.numpy as jnp
from jax import lax
from jax.experimental import pallas as pl
from jax.experimental.pallas import tpu as pltpu
from jax.experimental.pallas import tpu_sc as plsc

N = 10000
E = 320000
D = 128

NC = 2
NS = 16
NW = NC * NS
EPW = E // NW
K = 80
NCHUNK = EPW // K
RPT = N // NS
CW = 16
F32 = jnp.float32


def _make_sc_segsum(with_counts):
    mesh = plsc.VectorSubcoreMesh(core_axis_name="c", subcore_axis_name="s")
    out_type = [jax.ShapeDtypeStruct((NC, N, D), F32)]
    scratch = [
        pltpu.VMEM((K,), jnp.int32),
        pltpu.VMEM((K,), jnp.int32),
        pltpu.VMEM((K, D), F32),
        pltpu.VMEM((K, D), F32),
        pltpu.VMEM_SHARED((N, D), F32),
        pltpu.SemaphoreType.DMA,
    ]
    if with_counts:
        out_type.append(jax.ShapeDtypeStruct((NC, N, CW), F32))
        scratch += [
            pltpu.VMEM((K, CW), F32),
            pltpu.VMEM((K, CW), F32),
            pltpu.VMEM_SHARED((N, CW), F32),
        ]

    def body(x_hbm, src_hbm, dst_hbm, *refs):
        if with_counts:
            (p_out, c_out, src_v, dst_v, rows_v, zrow_v, acc, sem,
             ones_v, zcnt_v, accc) = refs
        else:
            (p_out, src_v, dst_v, rows_v, zrow_v, acc, sem) = refs
        cid = lax.axis_index("c")
        tid = lax.axis_index("s")
        wid = tid * NC + cid
        r0 = tid * RPT

        def fill_rowbuf(i, _):
            for j in range(D // 16):
                zrow_v[i, pl.ds(j * 16, 16)] = jnp.zeros((16,), F32)
            return 0
        lax.fori_loop(0, K, fill_rowbuf, 0)

        if with_counts:
            def fill_cnt(i, _):
                ones_v[i, pl.ds(0, 16)] = jnp.full((16,), 1.0, F32)
                zcnt_v[i, pl.ds(0, 16)] = jnp.zeros((16,), F32)
                return 0
            lax.fori_loop(0, K, fill_cnt, 0)

        nfull, rem = divmod(RPT, K)
        for q in range(nfull):
            pltpu.sync_copy(zrow_v, acc.at[pl.ds(r0 + q * K, K)])
            if with_counts:
                pltpu.sync_copy(zcnt_v, accc.at[pl.ds(r0 + q * K, K)])
        if rem:
            pltpu.sync_copy(zrow_v.at[pl.ds(0, rem)],
                            acc.at[pl.ds(r0 + nfull * K, rem)])
            if with_counts:
                pltpu.sync_copy(zcnt_v.at[pl.ds(0, rem)],
                                accc.at[pl.ds(r0 + nfull * K, rem)])
        plsc.subcore_barrier()

        base = wid * EPW

        def chunk(c, _):
            off = pl.multiple_of(base + c * K, 8)
            pltpu.sync_copy(src_hbm.at[pl.ds(off, K)], src_v)
            pltpu.sync_copy(dst_hbm.at[pl.ds(off, K)], dst_v)
            pltpu.async_copy(x_hbm.at[src_v], rows_v, sem).wait()
            pltpu.sync_copy(rows_v, acc.at[dst_v], add=True)
            if with_counts:
                pltpu.sync_copy(ones_v, accc.at[dst_v], add=True)
            return 0
        lax.fori_loop(0, NCHUNK, chunk, 0)

        plsc.subcore_barrier()
        pltpu.sync_copy(acc.at[pl.ds(r0, RPT)],
                        p_out.at[cid, pl.ds(r0, RPT)])
        if with_counts:
            pltpu.sync_copy(accc.at[pl.ds(r0, RPT)],
                            c_out.at[cid, pl.ds(r0, RPT)])

    return pl.kernel(body, out_type=out_type, mesh=mesh,
                     scratch_types=scratch)


_sc_segsum_counts = _make_sc_segsum(with_counts=True)
_sc_segsum = _make_sc_segsum(with_counts=False)

R = 500


def _tc_layer(x, P, C, Wn, Ws, b, relu):
    def body(p_ref, c_ref, x_ref, wn_ref, ws_ref, b_ref, o_ref):
        s = p_ref[0] + p_ref[1]
        cnt = c_ref[0, :, 0:1] + c_ref[1, :, 0:1]
        agg = s / jnp.maximum(cnt, 1.0)
        acc = (jnp.dot(agg, wn_ref[...], preferred_element_type=F32)
               + jnp.dot(x_ref[...], ws_ref[...], preferred_element_type=F32)
               + b_ref[...])
        if relu:
            acc = jnp.maximum(acc, 0.0)
        o_ref[...] = acc

    return pl.pallas_call(
        body,
        grid=(N // R,),
        in_specs=[
            pl.BlockSpec((2, R, D), lambda i: (0, i, 0)),
            pl.BlockSpec((2, R, CW), lambda i: (0, i, 0)),
            pl.BlockSpec((R, D), lambda i: (i, 0)),
            pl.BlockSpec((D, D), lambda i: (0, 0)),
            pl.BlockSpec((D, D), lambda i: (0, 0)),
            pl.BlockSpec((1, D), lambda i: (0, 0)),
        ],
        out_specs=pl.BlockSpec((R, D), lambda i: (i, 0)),
        out_shape=jax.ShapeDtypeStruct((N, D), F32),
    )(P, C, x, Wn, Ws, b.reshape(1, D))


def kernel(x, edge_index, W_neigh1, W_self1, b1, W_neigh2, W_self2, b2):
    src = edge_index[0]
    dst = edge_index[1]
    P1, C = _sc_segsum_counts(x, src, dst)
    h = _tc_layer(x, P1, C, W_neigh1, W_self1, b1, relu=True)
    (P2,) = _sc_segsum(h, src, dst)
    out = _tc_layer(h, P2, C, W_neigh2, W_self2, b2, relu=False)
    return out


# speedup vs baseline: 4.8638x; 4.8638x over previous
import jax
import jax.numpy as jnp
from jax import lax
from jax.experimental import pallas as pl
from jax.experimental.pallas import tpu as pltpu
from jax.experimental.pallas import tpu_sc as plsc

N = 10000
E = 320000
D = 128

NC = 2
NS = 16
NW = NC * NS
EPW = E // NW
K = 80
NCHUNK = EPW // K
NROWCH = N // K
QMAX = -(-NROWCH // NS)
CW = 128
F32 = jnp.float32

_MESH = dict(core_axis_name="c", subcore_axis_name="s")


def _worker_ids():
    cid = lax.axis_index("c")
    tid = lax.axis_index("s")
    return cid, tid, tid * NC + cid


def _for_owned_row_chunks(tid, fn):
    for q in range(QMAX):
        m = tid + NS * q

        @pl.when(m < NROWCH)
        def _(m=m):
            fn(pl.multiple_of(m * K, 8))


def _fill(ref, rows, width, val):
    def body(i, _):
        for j in range(width // 16):
            ref[i, pl.ds(j * 16, 16)] = jnp.full((16,), val, F32)
        return 0
    lax.fori_loop(0, rows, body, 0)


def _sc_segsum_body(x_hbm, src_hbm, dst_hbm, p_out,
                    src_v, dst_v, rows_v, zrow_v, acc, sem):
    cid, tid, wid = _worker_ids()

    _fill(zrow_v, K, D, 0.0)
    _for_owned_row_chunks(
        tid, lambda off: pltpu.sync_copy(zrow_v, acc.at[pl.ds(off, K)]))
    plsc.subcore_barrier()

    base = wid * EPW

    def chunk(c, _):
        off = pl.multiple_of(base + c * K, 8)
        pltpu.sync_copy(src_hbm.at[pl.ds(off, K)], src_v)
        pltpu.sync_copy(dst_hbm.at[pl.ds(off, K)], dst_v)
        pltpu.async_copy(x_hbm.at[src_v], rows_v, sem).wait()
        pltpu.sync_copy(rows_v, acc.at[dst_v], add=True)
        return 0
    lax.fori_loop(0, NCHUNK, chunk, 0)

    plsc.subcore_barrier()
    _for_owned_row_chunks(
        tid, lambda off: pltpu.sync_copy(acc.at[pl.ds(off, K)],
                                         p_out.at[cid, pl.ds(off, K)]))


_sc_segsum = pl.kernel(
    _sc_segsum_body,
    out_type=[jax.ShapeDtypeStruct((NC, N, D), F32)],
    mesh=plsc.VectorSubcoreMesh(**_MESH),
    scratch_types=[
        pltpu.VMEM((K,), jnp.int32),
        pltpu.VMEM((K,), jnp.int32),
        pltpu.VMEM((K, D), F32),
        pltpu.VMEM((K, D), F32),
        pltpu.VMEM_SHARED((N, D), F32),
        pltpu.SemaphoreType.DMA,
    ],
)


def _sc_counts_body(dst_hbm, c_out, dst_v, ones_v, zcnt_v, accc):
    cid, tid, wid = _worker_ids()

    _fill(ones_v, K, CW, 1.0)
    _fill(zcnt_v, K, CW, 0.0)
    _for_owned_row_chunks(
        tid, lambda off: pltpu.sync_copy(zcnt_v, accc.at[pl.ds(off, K)]))
    plsc.subcore_barrier()

    base = wid * EPW

    def chunk(c, _):
        off = pl.multiple_of(base + c * K, 8)
        pltpu.sync_copy(dst_hbm.at[pl.ds(off, K)], dst_v)
        pltpu.sync_copy(ones_v, accc.at[dst_v], add=True)
        return 0
    lax.fori_loop(0, NCHUNK, chunk, 0)

    plsc.subcore_barrier()
    _for_owned_row_chunks(
        tid, lambda off: pltpu.sync_copy(accc.at[pl.ds(off, K)],
                                         c_out.at[cid, pl.ds(off, K)]))


_sc_counts = pl.kernel(
    _sc_counts_body,
    out_type=[jax.ShapeDtypeStruct((NC, N, CW), F32)],
    mesh=plsc.VectorSubcoreMesh(**_MESH),
    scratch_types=[
        pltpu.VMEM((K,), jnp.int32),
        pltpu.VMEM((K, CW), F32),
        pltpu.VMEM((K, CW), F32),
        pltpu.VMEM_SHARED((N, CW), F32),
    ],
)

R = 1000


def _tc_layer(x, P, C, Wn, Ws, b, relu):
    def body(p_ref, c_ref, x_ref, wn_ref, ws_ref, b_ref, o_ref):
        s = p_ref[0] + p_ref[1]
        cnt = c_ref[0, :, 0:1] + c_ref[1, :, 0:1]
        agg = s / jnp.maximum(cnt, 1.0)
        acc = (jnp.dot(agg, wn_ref[...], preferred_element_type=F32)
               + jnp.dot(x_ref[...], ws_ref[...], preferred_element_type=F32)
               + b_ref[...])
        if relu:
            acc = jnp.maximum(acc, 0.0)
        o_ref[...] = acc

    return pl.pallas_call(
        body,
        grid=(N // R,),
        in_specs=[
            pl.BlockSpec((2, R, D), lambda i: (0, i, 0)),
            pl.BlockSpec((2, R, CW), lambda i: (0, i, 0)),
            pl.BlockSpec((R, D), lambda i: (i, 0)),
            pl.BlockSpec((D, D), lambda i: (0, 0)),
            pl.BlockSpec((D, D), lambda i: (0, 0)),
            pl.BlockSpec((1, D), lambda i: (0, 0)),
        ],
        out_specs=pl.BlockSpec((R, D), lambda i: (i, 0)),
        out_shape=jax.ShapeDtypeStruct((N, D), F32),
    )(P, C, x, Wn, Ws, b.reshape(1, D))


def kernel(x, edge_index, W_neigh1, W_self1, b1, W_neigh2, W_self2, b2):
    src = edge_index[0]
    dst = edge_index[1]
    (C,) = _sc_counts(dst)
    (P1,) = _sc_segsum(x, src, dst)
    h = _tc_layer(x, P1, C, W_neigh1, W_self1, b1, relu=True)
    (P2,) = _sc_segsum(h, src, dst)
    out = _tc_layer(h, P2, C, W_neigh2, W_self2, b2, relu=False)
    return out


# speedup vs baseline: 10.7175x; 2.2035x over previous
import jax
import jax.numpy as jnp
from jax import lax
from jax.experimental import pallas as pl
from jax.experimental.pallas import tpu as pltpu
from jax.experimental.pallas import tpu_sc as plsc

N = 10000
E = 320000
D = 128

NC = 2
NS = 16
NW = NC * NS
EPW = E // NW
K = 80
NCHUNK = EPW // K
NROWCH = N // K
QMAX = -(-NROWCH // NS)
CW = 128
F32 = jnp.float32

_MESH = dict(core_axis_name="c", subcore_axis_name="s")


def _worker_ids():
    cid = lax.axis_index("c")
    tid = lax.axis_index("s")
    return cid, tid, tid * NC + cid


def _for_owned_row_chunks(tid, fn):
    for q in range(QMAX):
        m = tid + NS * q

        @pl.when(m < NROWCH)
        def _(m=m):
            fn(pl.multiple_of(m * K, 8))


def _fill(ref, rows, width, val):
    def body(i, _):
        for j in range(width // 16):
            ref[i, pl.ds(j * 16, 16)] = jnp.full((16,), val, F32)
        return 0
    lax.fori_loop(0, rows, body, 0)


NBUF = 2
DBUF = 4
UNROLL = 4
NFULL = NCHUNK // UNROLL


def _sc_segsum_body(x_hbm, src_hbm, dst_hbm, p_out, src_all, *refs):
    rows = refs[:NBUF]
    sem_g = refs[NBUF:2 * NBUF]
    dbuf = refs[2 * NBUF:2 * NBUF + DBUF]
    sem_d = refs[2 * NBUF + DBUF:2 * NBUF + 2 * DBUF]
    acc = refs[2 * NBUF + 2 * DBUF]
    cid, tid, wid = _worker_ids()

    _fill(rows[0], K, D, 0.0)
    _for_owned_row_chunks(
        tid, lambda off: pltpu.sync_copy(rows[0], acc.at[pl.ds(off, K)]))
    plsc.subcore_barrier()

    pltpu.sync_copy(src_hbm.at[wid], src_all)
    base = wid * EPW

    def gather(c, j):
        pltpu.async_copy(x_hbm.at[src_all.at[c]], rows[j], sem_g[j])

    def dst_load(c, q):
        off = pl.multiple_of(base + c * K, 8)
        pltpu.async_copy(dst_hbm.at[pl.ds(off, K)], dbuf[q], sem_d[q])

    def wait_scatter(c, j, q):
        off = pl.multiple_of(base + c * K, 8)
        pltpu.make_async_copy(x_hbm.at[src_all.at[c]], rows[j],
                              sem_g[j]).wait()
        pltpu.make_async_copy(dst_hbm.at[pl.ds(off, K)], dbuf[q],
                              sem_d[q]).wait()
        pltpu.sync_copy(rows[j], acc.at[dbuf[q]], add=True)

    for q in range(DBUF - 1):
        dst_load(q, q)
    for j in range(NBUF):
        gather(j, j)

    def group(i, _):
        for u in range(UNROLL):
            c = i * UNROLL + u
            j = u % NBUF
            q = u % DBUF
            @pl.when(c + DBUF - 1 < NCHUNK)
            def _():
                dst_load(c + DBUF - 1, (u + DBUF - 1) % DBUF)

            wait_scatter(c, j, q)

            @pl.when(c + NBUF < NCHUNK)
            def _():
                gather(c + NBUF, j)
        return 0
    lax.fori_loop(0, NFULL, group, 0)

    for t in range(NCHUNK % UNROLL):
        wait_scatter(NFULL * UNROLL + t, t % NBUF, t % DBUF)

    plsc.subcore_barrier()
    _for_owned_row_chunks(
        tid, lambda off: pltpu.sync_copy(acc.at[pl.ds(off, K)],
                                         p_out.at[cid, pl.ds(off, K)]))


_sc_segsum = pl.kernel(
    _sc_segsum_body,
    out_type=[jax.ShapeDtypeStruct((NC, N, D), F32)],
    mesh=plsc.VectorSubcoreMesh(**_MESH),
    scratch_types=[
        pltpu.VMEM((NCHUNK, K), jnp.int32),
        *[pltpu.VMEM((K, D), F32) for _ in range(NBUF)],
        *[pltpu.SemaphoreType.DMA for _ in range(NBUF)],
        *[pltpu.VMEM((K,), jnp.int32) for _ in range(DBUF)],
        *[pltpu.SemaphoreType.DMA for _ in range(DBUF)],
        pltpu.VMEM_SHARED((N, D), F32),
    ],
)


def _sc_counts_body(dst_hbm, c_out, dst_all, ones_v, zcnt_v, accc):
    cid, tid, wid = _worker_ids()

    _fill(ones_v, K, CW, 1.0)
    _fill(zcnt_v, K, CW, 0.0)
    _for_owned_row_chunks(
        tid, lambda off: pltpu.sync_copy(zcnt_v, accc.at[pl.ds(off, K)]))
    plsc.subcore_barrier()

    pltpu.sync_copy(dst_hbm.at[wid], dst_all)

    def chunk(c, _):
        pltpu.sync_copy(ones_v, accc.at[dst_all.at[c]], add=True)
        return 0
    lax.fori_loop(0, NCHUNK, chunk, 0)

    plsc.subcore_barrier()
    _for_owned_row_chunks(
        tid, lambda off: pltpu.sync_copy(accc.at[pl.ds(off, K)],
                                         c_out.at[cid, pl.ds(off, K)]))


_sc_counts = pl.kernel(
    _sc_counts_body,
    out_type=[jax.ShapeDtypeStruct((NC, N, CW), F32)],
    mesh=plsc.VectorSubcoreMesh(**_MESH),
    scratch_types=[
        pltpu.VMEM((NCHUNK, K), jnp.int32),
        pltpu.VMEM((K, CW), F32),
        pltpu.VMEM((K, CW), F32),
        pltpu.VMEM_SHARED((N, CW), F32),
    ],
)

R = 1000


def _tc_layer(x, P, C, Wn, Ws, b, relu):
    def body(p_ref, c_ref, x_ref, wn_ref, ws_ref, b_ref, o_ref):
        s = p_ref[0] + p_ref[1]
        cnt = c_ref[0, :, 0:1] + c_ref[1, :, 0:1]
        agg = s / jnp.maximum(cnt, 1.0)
        acc = (jnp.dot(agg, wn_ref[...], preferred_element_type=F32)
               + jnp.dot(x_ref[...], ws_ref[...], preferred_element_type=F32)
               + b_ref[...])
        if relu:
            acc = jnp.maximum(acc, 0.0)
        o_ref[...] = acc

    return pl.pallas_call(
        body,
        grid=(N // R,),
        in_specs=[
            pl.BlockSpec((2, R, D), lambda i: (0, i, 0)),
            pl.BlockSpec((2, R, CW), lambda i: (0, i, 0)),
            pl.BlockSpec((R, D), lambda i: (i, 0)),
            pl.BlockSpec((D, D), lambda i: (0, 0)),
            pl.BlockSpec((D, D), lambda i: (0, 0)),
            pl.BlockSpec((1, D), lambda i: (0, 0)),
        ],
        out_specs=pl.BlockSpec((R, D), lambda i: (i, 0)),
        out_shape=jax.ShapeDtypeStruct((N, D), F32),
    )(P, C, x, Wn, Ws, b.reshape(1, D))


def kernel(x, edge_index, W_neigh1, W_self1, b1, W_neigh2, W_self2, b2):
    src3 = edge_index[0].reshape(NW, NCHUNK, K)
    dst3 = edge_index[1].reshape(NW, NCHUNK, K)
    dst1 = edge_index[1]
    (C,) = _sc_counts(dst3)
    (P1,) = _sc_segsum(x, src3, dst1)
    h = _tc_layer(x, P1, C, W_neigh1, W_self1, b1, relu=True)
    (P2,) = _sc_segsum(h, src3, dst1)
    out = _tc_layer(h, P2, C, W_neigh2, W_self2, b2, relu=False)
    return out


# speedup vs baseline: 12.1186x; 1.1307x over previous
import jax
import jax.numpy as jnp
from jax import lax
from jax.experimental import pallas as pl
from jax.experimental.pallas import tpu as pltpu
from jax.experimental.pallas import tpu_sc as plsc

N = 10000
E = 320000
D = 128

NC = 2
NS = 16
NW = NC * NS
EPW = E // NW
K = 80
NCHUNK = EPW // K
NROWCH = N // K
QMAX = -(-NROWCH // NS)
CW = 128
F32 = jnp.float32

_MESH = dict(core_axis_name="c", subcore_axis_name="s")


def _worker_ids():
    cid = lax.axis_index("c")
    tid = lax.axis_index("s")
    return cid, tid, tid * NC + cid


def _for_owned_row_chunks(tid, fn):
    for q in range(QMAX):
        m = tid + NS * q

        @pl.when(m < NROWCH)
        def _(m=m):
            fn(pl.multiple_of(m * K, 8))


def _fill(ref, rows, width, val):
    def body(i, _):
        for j in range(width // 16):
            ref[i, pl.ds(j * 16, 16)] = jnp.full((16,), val, F32)
        return 0
    lax.fori_loop(0, rows, body, 0)


NBUF = 3
IBUF = 4
UNROLL = 12
NFULL = NCHUNK // UNROLL
NTAIL = NCHUNK % UNROLL


def _sc_segsum_body(x_hbm, src_hbm, dst_hbm, p_out, *refs):
    rows = refs[:NBUF]
    sem_g = refs[NBUF:2 * NBUF]
    o = 2 * NBUF
    sbuf = refs[o:o + IBUF]
    sem_s = refs[o + IBUF:o + 2 * IBUF]
    dbuf = refs[o + 2 * IBUF:o + 3 * IBUF]
    sem_d = refs[o + 3 * IBUF:o + 4 * IBUF]
    acc = refs[o + 4 * IBUF]
    cid, tid, wid = _worker_ids()

    _fill(rows[0], K, D, 0.0)
    _for_owned_row_chunks(
        tid, lambda off: pltpu.sync_copy(rows[0], acc.at[pl.ds(off, K)]))
    plsc.subcore_barrier()

    base = wid * EPW

    def idx_load(c, q):
        off = pl.multiple_of(base + c * K, 8)
        pltpu.async_copy(src_hbm.at[pl.ds(off, K)], sbuf[q], sem_s[q])
        pltpu.async_copy(dst_hbm.at[pl.ds(off, K)], dbuf[q], sem_d[q])

    def gather(c, j, q):
        off = pl.multiple_of(base + c * K, 8)
        pltpu.make_async_copy(src_hbm.at[pl.ds(off, K)], sbuf[q],
                              sem_s[q]).wait()
        pltpu.async_copy(x_hbm.at[sbuf[q]], rows[j], sem_g[j])

    def wait_scatter(c, j, q):
        off = pl.multiple_of(base + c * K, 8)
        pltpu.make_async_copy(x_hbm.at[sbuf[q]], rows[j], sem_g[j]).wait()
        pltpu.make_async_copy(dst_hbm.at[pl.ds(off, K)], dbuf[q],
                              sem_d[q]).wait()
        pltpu.sync_copy(rows[j], acc.at[dbuf[q]], add=True)

    for q in range(IBUF):
        idx_load(q, q)
    for j in range(NBUF):
        gather(j, j, j)

    def group(i, _):
        for u in range(UNROLL):
            c = i * UNROLL + u
            j = u % NBUF
            q = u % IBUF
            wait_scatter(c, j, q)
            idx_load(c + IBUF, q)
            gather(c + NBUF, j, (u + NBUF) % IBUF)
        return 0
    lax.fori_loop(0, NFULL, group, 0)

    for t in range(NTAIL):
        c = NFULL * UNROLL + t
        wait_scatter(c, t % NBUF, t % IBUF)
        if c + IBUF < NCHUNK:
            idx_load(c + IBUF, t % IBUF)
        if c + NBUF < NCHUNK:
            gather(c + NBUF, (c + NBUF) % NBUF, (t + NBUF) % IBUF)

    plsc.subcore_barrier()
    _for_owned_row_chunks(
        tid, lambda off: pltpu.sync_copy(acc.at[pl.ds(off, K)],
                                         p_out.at[cid, pl.ds(off, K)]))


_sc_segsum = pl.kernel(
    _sc_segsum_body,
    out_type=[jax.ShapeDtypeStruct((NC, N, D), F32)],
    mesh=plsc.VectorSubcoreMesh(**_MESH),
    scratch_types=[
        *[pltpu.VMEM((K, D), F32) for _ in range(NBUF)],
        *[pltpu.SemaphoreType.DMA for _ in range(NBUF)],
        *[pltpu.VMEM((K,), jnp.int32) for _ in range(IBUF)],
        *[pltpu.SemaphoreType.DMA for _ in range(IBUF)],
        *[pltpu.VMEM((K,), jnp.int32) for _ in range(IBUF)],
        *[pltpu.SemaphoreType.DMA for _ in range(IBUF)],
        pltpu.VMEM_SHARED((N, D), F32),
    ],
)


def _make_sc_counts(cw):
    def body(dst_hbm, c_out, dst_all, ones_v, zcnt_v, accc):
        cid, tid, wid = _worker_ids()

        _fill(ones_v, K, cw, 1.0)
        _fill(zcnt_v, K, cw, 0.0)
        _for_owned_row_chunks(
            tid, lambda off: pltpu.sync_copy(zcnt_v, accc.at[pl.ds(off, K)]))
        plsc.subcore_barrier()

        pltpu.sync_copy(dst_hbm.at[wid], dst_all)

        def chunk(c, _):
            pltpu.sync_copy(ones_v, accc.at[dst_all.at[c]], add=True)
            return 0
        lax.fori_loop(0, NCHUNK, chunk, 0)

        plsc.subcore_barrier()
        _for_owned_row_chunks(
            tid, lambda off: pltpu.sync_copy(accc.at[pl.ds(off, K)],
                                             c_out.at[cid, pl.ds(off, K)]))

    return pl.kernel(
        body,
        out_type=[jax.ShapeDtypeStruct((NC, N, cw), F32)],
        mesh=plsc.VectorSubcoreMesh(**_MESH),
        scratch_types=[
            pltpu.VMEM((NCHUNK, K), jnp.int32),
            pltpu.VMEM((K, cw), F32),
            pltpu.VMEM((K, cw), F32),
            pltpu.VMEM_SHARED((N, cw), F32),
        ],
    )


_sc_counts = _make_sc_counts(CW)

R = 1000


def _tc_layer(x, P, C, Wn, Ws, b, relu):
    def body(p_ref, c_ref, x_ref, wn_ref, ws_ref, b_ref, o_ref):
        s = p_ref[0] + p_ref[1]
        cnt = c_ref[0, :, 0:1] + c_ref[1, :, 0:1]
        agg = s / jnp.maximum(cnt, 1.0)
        acc = (jnp.dot(agg, wn_ref[...], preferred_element_type=F32)
               + jnp.dot(x_ref[...], ws_ref[...], preferred_element_type=F32)
               + b_ref[...])
        if relu:
            acc = jnp.maximum(acc, 0.0)
        o_ref[...] = acc

    return pl.pallas_call(
        body,
        grid=(N // R,),
        in_specs=[
            pl.BlockSpec((2, R, D), lambda i: (0, i, 0)),
            pl.BlockSpec((2, R, CW), lambda i: (0, i, 0)),
            pl.BlockSpec((R, D), lambda i: (i, 0)),
            pl.BlockSpec((D, D), lambda i: (0, 0)),
            pl.BlockSpec((D, D), lambda i: (0, 0)),
            pl.BlockSpec((1, D), lambda i: (0, 0)),
        ],
        out_specs=pl.BlockSpec((R, D), lambda i: (i, 0)),
        out_shape=jax.ShapeDtypeStruct((N, D), F32),
    )(P, C, x, Wn, Ws, b.reshape(1, D))


def kernel(x, edge_index, W_neigh1, W_self1, b1, W_neigh2, W_self2, b2):
    src1 = edge_index[0]
    dst1 = edge_index[1]
    dst3 = dst1.reshape(NW, NCHUNK, K)
    (C,) = _sc_counts(dst3)
    (P1,) = _sc_segsum(x, src1, dst1)
    h = _tc_layer(x, P1, C, W_neigh1, W_self1, b1, relu=True)
    (P2,) = _sc_segsum(h, src1, dst1)
    out = _tc_layer(h, P2, C, W_neigh2, W_self2, b2, relu=False)
    return out


# speedup vs baseline: 12.3396x; 1.0182x over previous
import math

import jax
import jax.numpy as jnp
from jax import lax
from jax.experimental import pallas as pl
from jax.experimental.pallas import tpu as pltpu
from jax.experimental.pallas import tpu_sc as plsc

N = 10000
E = 320000
D = 128

NC = 2
NS = 16
NW = NC * NS
EPW = E // NW
K = 80
NCHUNK = EPW // K
NROWCH = N // K
QMAX = -(-NROWCH // NS)
F32 = jnp.float32

_MESH = dict(core_axis_name="c", subcore_axis_name="s")


def _worker_ids():
    cid = lax.axis_index("c")
    tid = lax.axis_index("s")
    return cid, tid, tid * NC + cid


def _for_owned_row_chunks(tid, fn):
    for q in range(QMAX):
        m = tid + NS * q

        @pl.when(m < NROWCH)
        def _(m=m):
            fn(pl.multiple_of(m * K, 8))


def _fill(ref, rows, width, val):
    def body(i, _):
        for j in range(width // 16):
            ref[i, pl.ds(j * 16, 16)] = jnp.full((16,), val, F32)
        return 0
    lax.fori_loop(0, rows, body, 0)


IBUF = 4


def _make_sc_segsum(width, nbuf):
    unroll = nbuf * IBUF // math.gcd(nbuf, IBUF)
    nfull = NCHUNK // unroll
    ntail = NCHUNK % unroll

    def body(x_hbm, src_hbm, dst_hbm, p_out, *refs):
        rows = refs[:nbuf]
        sem_g = refs[nbuf:2 * nbuf]
        o = 2 * nbuf
        sbuf = refs[o:o + IBUF]
        sem_s = refs[o + IBUF:o + 2 * IBUF]
        dbuf = refs[o + 2 * IBUF:o + 3 * IBUF]
        sem_d = refs[o + 3 * IBUF:o + 4 * IBUF]
        acc = refs[o + 4 * IBUF]
        cid, tid, wid = _worker_ids()

        _fill(rows[0], K, width, 0.0)
        _for_owned_row_chunks(
            tid, lambda off: pltpu.sync_copy(rows[0], acc.at[pl.ds(off, K)]))
        plsc.subcore_barrier()

        base = wid * EPW

        def idx_load(c, q):
            off = pl.multiple_of(base + c * K, 8)
            pltpu.async_copy(src_hbm.at[pl.ds(off, K)], sbuf[q], sem_s[q])
            pltpu.async_copy(dst_hbm.at[pl.ds(off, K)], dbuf[q], sem_d[q])

        def gather(c, j, q):
            off = pl.multiple_of(base + c * K, 8)
            pltpu.make_async_copy(src_hbm.at[pl.ds(off, K)], sbuf[q],
                                  sem_s[q]).wait()
            pltpu.async_copy(x_hbm.at[sbuf[q]], rows[j], sem_g[j])

        def wait_scatter(c, j, q):
            off = pl.multiple_of(base + c * K, 8)
            pltpu.make_async_copy(x_hbm.at[sbuf[q]], rows[j], sem_g[j]).wait()
            pltpu.make_async_copy(dst_hbm.at[pl.ds(off, K)], dbuf[q],
                                  sem_d[q]).wait()
            pltpu.sync_copy(rows[j], acc.at[dbuf[q]], add=True)

        for q in range(IBUF):
            idx_load(q, q)
        for j in range(nbuf):
            gather(j, j, j)

        def group(i, _):
            for u in range(unroll):
                c = i * unroll + u
                j = u % nbuf
                q = u % IBUF
                wait_scatter(c, j, q)
                idx_load(c + IBUF, q)
                gather(c + nbuf, j, (u + nbuf) % IBUF)
            return 0
        lax.fori_loop(0, nfull, group, 0)

        for t in range(ntail):
            c = nfull * unroll + t
            wait_scatter(c, t % nbuf, t % IBUF)
            if c + IBUF < NCHUNK:
                idx_load(c + IBUF, t % IBUF)
            if c + nbuf < NCHUNK:
                gather(c + nbuf, (c + nbuf) % nbuf, (t + nbuf) % IBUF)

        plsc.subcore_barrier()
        _for_owned_row_chunks(
            tid, lambda off: pltpu.sync_copy(acc.at[pl.ds(off, K)],
                                             p_out.at[cid, pl.ds(off, K)]))

    return pl.kernel(
        body,
        out_type=[jax.ShapeDtypeStruct((NC, N, width), F32)],
        mesh=plsc.VectorSubcoreMesh(**_MESH),
        scratch_types=[
            *[pltpu.VMEM((K, width), F32) for _ in range(nbuf)],
            *[pltpu.SemaphoreType.DMA for _ in range(nbuf)],
            *[pltpu.VMEM((K,), jnp.int32) for _ in range(IBUF)],
            *[pltpu.SemaphoreType.DMA for _ in range(IBUF)],
            *[pltpu.VMEM((K,), jnp.int32) for _ in range(IBUF)],
            *[pltpu.SemaphoreType.DMA for _ in range(IBUF)],
            pltpu.VMEM_SHARED((N, width), F32),
        ],
    )


_sc_segsum = _make_sc_segsum(D, 3)


CW = 128


def _sc_counts_body(dst_hbm, c_out, dst_all, ones_v, zcnt_v, accc):
    cid, tid, wid = _worker_ids()

    _fill(ones_v, K, CW, 1.0)
    _fill(zcnt_v, K, CW, 0.0)
    _for_owned_row_chunks(
        tid, lambda off: pltpu.sync_copy(zcnt_v, accc.at[pl.ds(off, K)]))
    plsc.subcore_barrier()

    pltpu.sync_copy(dst_hbm.at[wid], dst_all)

    def chunk(c, _):
        pltpu.sync_copy(ones_v, accc.at[dst_all.at[c]], add=True)
        return 0
    lax.fori_loop(0, NCHUNK, chunk, 0)

    plsc.subcore_barrier()
    _for_owned_row_chunks(
        tid, lambda off: pltpu.sync_copy(accc.at[pl.ds(off, K)],
                                         c_out.at[cid, pl.ds(off, K)]))


_sc_counts = pl.kernel(
    _sc_counts_body,
    out_type=[jax.ShapeDtypeStruct((NC, N, CW), F32)],
    mesh=plsc.VectorSubcoreMesh(**_MESH),
    scratch_types=[
        pltpu.VMEM((NCHUNK, K), jnp.int32),
        pltpu.VMEM((K, CW), F32),
        pltpu.VMEM((K, CW), F32),
        pltpu.VMEM_SHARED((N, CW), F32),
    ],
)


R = 2000


def _tc_layer(x, P, C, Wn, Ws, b, relu):
    cw = C.shape[2]

    def body(p_ref, c_ref, x_ref, wn_ref, ws_ref, b_ref, o_ref):
        s = p_ref[0] + p_ref[1]
        cnt = c_ref[0, :, 0:1] + c_ref[1, :, 0:1]
        agg = s / jnp.maximum(cnt, 1.0)
        acc = (jnp.dot(agg, wn_ref[...], preferred_element_type=F32)
               + jnp.dot(x_ref[...], ws_ref[...], preferred_element_type=F32)
               + b_ref[...])
        if relu:
            acc = jnp.maximum(acc, 0.0)
        o_ref[...] = acc

    return pl.pallas_call(
        body,
        grid=(N // R,),
        in_specs=[
            pl.BlockSpec((2, R, D), lambda i: (0, i, 0)),
            pl.BlockSpec((2, R, cw), lambda i: (0, i, 0)),
            pl.BlockSpec((R, D), lambda i: (i, 0)),
            pl.BlockSpec((D, D), lambda i: (0, 0)),
            pl.BlockSpec((D, D), lambda i: (0, 0)),
            pl.BlockSpec((1, D), lambda i: (0, 0)),
        ],
        out_specs=pl.BlockSpec((R, D), lambda i: (i, 0)),
        out_shape=jax.ShapeDtypeStruct((N, D), F32),
    )(P, C, x, Wn, Ws, b.reshape(1, D))


def kernel(x, edge_index, W_neigh1, W_self1, b1, W_neigh2, W_self2, b2):
    src1 = edge_index[0]
    dst1 = edge_index[1]
    dst3 = dst1.reshape(NW, NCHUNK, K)
    (C,) = _sc_counts(dst3)
    (P1,) = _sc_segsum(x, src1, dst1)
    h = _tc_layer(x, P1, C, W_neigh1, W_self1, b1, relu=True)
    (P2,) = _sc_segsum(h, src1, dst1)
    out = _tc_layer(h, P2, C, W_neigh2, W_self2, b2, relu=False)
    return out
